# baseline (speedup 1.0000x reference)
import functools
import jax
import jax.numpy as jnp
from jax import lax
from jax.experimental import pallas as pl
from jax.experimental.pallas import tpu as pltpu
from jax.experimental.pallas import tpu_sc as plsc

D_MODEL = 768
N_EXP = 8
D_EXP = 2048
T = 2048
TM = 128
NT = 40
NSLOT = NT * TM
DAUG = D_MODEL + 128
NW = 32
TPW = T // NW
CHUNK = 256


def _router_body(x_ref, wg_ref, xaug_ref, slots_ref, te_ref, tbnt_ref,
                 m_scr, rank_scr):
    x = x_ref[...]
    logits = jnp.dot(x, wg_ref[...], preferred_element_type=jnp.float32)
    cols = jax.lax.broadcasted_iota(jnp.int32, logits.shape, 1)
    big = jnp.int32(N_EXP)
    v1 = jnp.max(logits, axis=1, keepdims=True)
    i1 = jnp.min(jnp.where(logits == v1, cols, big), axis=1, keepdims=True)
    l2 = jnp.where(cols == i1, -jnp.inf, logits)
    v2 = jnp.max(l2, axis=1, keepdims=True)
    i2 = jnp.min(jnp.where(l2 == v2, cols, big), axis=1, keepdims=True)
    e2 = jnp.exp(v2 - v1)
    denom = 1.0 + e2
    g1 = 1.0 / denom
    g2 = e2 / denom

    m = ((cols == i1) | (cols == i2)).astype(jnp.float32)
    m_scr[...] = m

    ri = jax.lax.broadcasted_iota(jnp.int32, (CHUNK, CHUNK), 0)
    ci = jax.lax.broadcasted_iota(jnp.int32, (CHUNK, CHUNK), 1)
    ltri = (ri > ci).astype(jnp.float32)

    def chunk_body(c, carry):
        mc = m_scr[pl.ds(c * CHUNK, CHUNK), :]
        rank_scr[pl.ds(c * CHUNK, CHUNK), :] = (
            jnp.dot(ltri, mc, preferred_element_type=jnp.float32) + carry)
        return carry + jnp.sum(mc, axis=0, keepdims=True)

    counts = jax.lax.fori_loop(0, T // CHUNK, chunk_body,
                               jnp.zeros((1, N_EXP), jnp.float32))

    ntiles = jnp.floor((counts + (TM - 1)) / TM)
    ei = jax.lax.broadcasted_iota(jnp.int32, (N_EXP, N_EXP), 0)
    ej = jax.lax.broadcasted_iota(jnp.int32, (N_EXP, N_EXP), 1)
    strict = (ei < ej).astype(jnp.float32)
    tile_base = jnp.dot(ntiles, strict, preferred_element_type=jnp.float32)
    offset = tile_base * TM

    slot_full = offset + rank_scr[...]
    slot1 = jnp.sum(jnp.where(cols == i1, slot_full, 0.0), axis=1, keepdims=True)
    slot2 = jnp.sum(jnp.where(cols == i2, slot_full, 0.0), axis=1, keepdims=True)

    sc = jax.lax.broadcasted_iota(jnp.int32, (T, 128), 1)
    s1b = jnp.broadcast_to(slot1, (T, 128))
    s2b = jnp.broadcast_to(slot2, (T, 128))
    slots_ref[...] = jnp.where(sc == 0, s1b, jnp.where(sc == 1, s2b, 0.0)
                               ).astype(jnp.int32)

    tj = jax.lax.broadcasted_iota(jnp.int32, (NT, N_EXP), 0).astype(jnp.float32)
    eid = jax.lax.broadcasted_iota(jnp.int32, (NT, N_EXP), 1).astype(jnp.float32)
    tb = jnp.broadcast_to(tile_base, (NT, N_EXP))
    ntb = jnp.broadcast_to(ntiles, (NT, N_EXP))
    ind = ((tj >= tb) & (tj < tb + ntb)).astype(jnp.float32)
    te = jnp.sum(ind * eid, axis=1, keepdims=True)
    te_ref[...] = jnp.broadcast_to(te, (NT, 128)).astype(jnp.int32)

    ri16 = jax.lax.broadcasted_iota(jnp.int32, (16, N_EXP), 0)
    ci16 = jax.lax.broadcasted_iota(jnp.int32, (16, N_EXP), 1)
    tbb = jnp.broadcast_to(tile_base, (16, N_EXP))
    ntbb = jnp.broadcast_to(ntiles, (16, N_EXP))
    isbase = (ri16 < 8).astype(jnp.float32)
    sel = tbb * isbase + ntbb * (1.0 - isbase)
    diag = (ci16 == (ri16 & 7)).astype(jnp.float32)
    tbnt = jnp.sum(diag * sel, axis=1, keepdims=True)
    tbnt_ref[...] = jnp.broadcast_to(tbnt, (16, 128)).astype(jnp.int32)

    xaug_ref[0] = jnp.concatenate(
        [x, jnp.broadcast_to(g1, (T, 128))], axis=1)
    xaug_ref[1] = jnp.concatenate(
        [x, jnp.broadcast_to(g2, (T, 128))], axis=1)


@jax.jit
def _router(x2d, w_gate):
    return pl.pallas_call(
        _router_body,
        in_specs=[pl.BlockSpec((T, D_MODEL), lambda: (0, 0)),
                  pl.BlockSpec((D_MODEL, N_EXP), lambda: (0, 0))],
        out_specs=[pl.BlockSpec((2, T, DAUG), lambda: (0, 0, 0)),
                   pl.BlockSpec((T, 128), lambda: (0, 0)),
                   pl.BlockSpec((NT, 128), lambda: (0, 0)),
                   pl.BlockSpec((16, 128), lambda: (0, 0))],
        out_shape=[jax.ShapeDtypeStruct((2, T, DAUG), jnp.float32),
                   jax.ShapeDtypeStruct((T, 128), jnp.int32),
                   jax.ShapeDtypeStruct((NT, 128), jnp.int32),
                   jax.ShapeDtypeStruct((16, 128), jnp.int32)],
        scratch_shapes=[pltpu.VMEM((T, N_EXP), jnp.float32),
                        pltpu.VMEM((T, N_EXP), jnp.float32)],
    )(x2d, w_gate)


@functools.cache
def _sc_dispatch():
    mesh = plsc.VectorSubcoreMesh(core_axis_name="c", subcore_axis_name="s")

    @functools.partial(
        pl.kernel, mesh=mesh,
        out_type=jax.ShapeDtypeStruct((NSLOT, DAUG), jnp.float32),
        scratch_types=[pltpu.VMEM((TPW,), jnp.int32),
                       pltpu.VMEM((TPW,), jnp.int32),
                       pltpu.VMEM((TPW, DAUG), jnp.float32),
                       pltpu.VMEM((TPW, DAUG), jnp.float32),
                       pltpu.SemaphoreType.DMA,
                       pltpu.SemaphoreType.DMA],
    )
    def _dispatch(xaug_hbm, slot1_hbm, slot2_hbm, xg_hbm, idx1_v, idx2_v,
                  buf1_v, buf2_v, sem1, sem2):
        wid = lax.axis_index("s") * 2 + lax.axis_index("c")
        base = wid * TPW
        pltpu.sync_copy(slot1_hbm.at[wid], idx1_v)
        pltpu.sync_copy(xaug_hbm.at[0, pl.ds(base, TPW)], buf1_v)
        cp1 = pltpu.async_copy(buf1_v, xg_hbm.at[idx1_v], sem1)
        pltpu.sync_copy(slot2_hbm.at[wid], idx2_v)
        pltpu.sync_copy(xaug_hbm.at[1, pl.ds(base, TPW)], buf2_v)
        cp2 = pltpu.async_copy(buf2_v, xg_hbm.at[idx2_v], sem2)
        cp1.wait()
        cp2.wait()

    return _dispatch


def _w_copies(w1_hbm, w2_hbm, w1_buf, w2_buf, wsem, e, slot):
    half = D_EXP // 2
    return [
        pltpu.make_async_copy(w1_hbm.at[e, :, pl.ds(0, half)],
                              w1_buf.at[slot, :, pl.ds(0, half)], wsem),
        pltpu.make_async_copy(w1_hbm.at[e, :, pl.ds(half, half)],
                              w1_buf.at[slot, :, pl.ds(half, half)], wsem),
        pltpu.make_async_copy(w2_hbm.at[e, pl.ds(0, half)],
                              w2_buf.at[slot, pl.ds(0, half)], wsem),
        pltpu.make_async_copy(w2_hbm.at[e, pl.ds(half, half)],
                              w2_buf.at[slot, pl.ds(half, half)], wsem),
    ]


def _gmm_body(tbnt_ref, xg_hbm, w1_hbm, b1_hbm, w2_hbm, b2_hbm, y_hbm,
              w1_buf, w2_buf, b1_buf, b2_buf, xg_buf, y_buf,
              wsems, bsem, xsem, ysem):
    pltpu.make_async_copy(b1_hbm, b1_buf, bsem).start()
    pltpu.make_async_copy(b2_hbm, b2_buf, bsem).start()

    def fetch(e, slot):
        @pl.when(tbnt_ref[8 + e] > 0)
        def _():
            for cp in _w_copies(w1_hbm, w2_hbm, w1_buf, w2_buf,
                                wsems.at[slot], e, slot):
                cp.start()

    fetch(0, 0)
    pltpu.make_async_copy(b1_hbm, b1_buf, bsem).wait()
    pltpu.make_async_copy(b2_hbm, b2_buf, bsem).wait()

    for e in range(N_EXP):
        slot = e % 2
        if e + 1 < N_EXP:
            fetch(e + 1, (e + 1) % 2)
        nt_e = tbnt_ref[8 + e]
        tb_e = tbnt_ref[e]

        @pl.when(nt_e > 0)
        def _(e=e, slot=slot, nt_e=nt_e, tb_e=tb_e):
            for cp in _w_copies(w1_hbm, w2_hbm, w1_buf, w2_buf,
                                wsems.at[slot], e, slot):
                cp.wait()
            w1 = w1_buf[slot]
            w2 = w2_buf[slot]
            b1 = b1_buf[e, :].reshape(1, D_EXP)
            b2 = b2_buf[e, :].reshape(1, D_MODEL)

            def tile_body(k, _):
                j = tb_e + k
                rows = pl.ds(j * TM, TM)
                pltpu.make_async_copy(xg_hbm.at[rows], xg_buf, xsem).start()
                pltpu.make_async_copy(xg_hbm.at[rows], xg_buf, xsem).wait()
                xg = xg_buf[...]
                x = xg[:, :D_MODEL]
                g = xg[:, D_MODEL:D_MODEL + 1]
                h = jnp.maximum(
                    jnp.dot(x, w1, preferred_element_type=jnp.float32) + b1,
                    0.0)
                y_buf[...] = (jnp.dot(h, w2, preferred_element_type=jnp.float32)
                              + b2) * g
                pltpu.make_async_copy(y_buf, y_hbm.at[rows], ysem).start()
                pltpu.make_async_copy(y_buf, y_hbm.at[rows], ysem).wait()
                return 0

            jax.lax.fori_loop(0, nt_e, tile_body, 0)


@jax.jit
def _gmm(tbnt, xg, w1, b1, w2, b2):
    grid_spec = pltpu.PrefetchScalarGridSpec(
        num_scalar_prefetch=1,
        grid=(1,),
        in_specs=[
            pl.BlockSpec(memory_space=pl.ANY),
            pl.BlockSpec(memory_space=pl.ANY),
            pl.BlockSpec(memory_space=pl.ANY),
            pl.BlockSpec(memory_space=pl.ANY),
            pl.BlockSpec(memory_space=pl.ANY),
        ],
        out_specs=pl.BlockSpec(memory_space=pl.ANY),
        scratch_shapes=[
            pltpu.VMEM((2, D_MODEL, D_EXP), jnp.float32),
            pltpu.VMEM((2, D_EXP, D_MODEL), jnp.float32),
            pltpu.VMEM((N_EXP, D_EXP), jnp.float32),
            pltpu.VMEM((N_EXP, D_MODEL), jnp.float32),
            pltpu.VMEM((TM, DAUG), jnp.float32),
            pltpu.VMEM((TM, D_MODEL), jnp.float32),
            pltpu.SemaphoreType.DMA((2,)),
            pltpu.SemaphoreType.DMA,
            pltpu.SemaphoreType.DMA,
            pltpu.SemaphoreType.DMA,
        ],
    )
    return pl.pallas_call(
        _gmm_body,
        grid_spec=grid_spec,
        out_shape=jax.ShapeDtypeStruct((NSLOT, D_MODEL), jnp.float32),
    )(tbnt, xg, w1, b1, w2, b2)


@functools.cache
def _sc_combine():
    mesh = plsc.VectorSubcoreMesh(core_axis_name="c", subcore_axis_name="s")

    @functools.partial(
        pl.kernel, mesh=mesh,
        out_type=jax.ShapeDtypeStruct((T, D_MODEL), jnp.float32),
        scratch_types=[pltpu.VMEM((TPW,), jnp.int32),
                       pltpu.VMEM((TPW,), jnp.int32),
                       pltpu.VMEM((TPW, D_MODEL), jnp.float32),
                       pltpu.VMEM((TPW, D_MODEL), jnp.float32),
                       pltpu.SemaphoreType.DMA],
    )
    def _combine(y_hbm, slot1_hbm, slot2_hbm, out_hbm, i1_v, i2_v, b1_v, b2_v,
                 sem):
        wid = lax.axis_index("s") * 2 + lax.axis_index("c")
        base = wid * TPW
        pltpu.sync_copy(slot1_hbm.at[wid], i1_v)
        pltpu.sync_copy(slot2_hbm.at[wid], i2_v)
        pltpu.async_copy(y_hbm.at[i1_v], b1_v, sem).wait()
        pltpu.async_copy(y_hbm.at[i2_v], b2_v, sem).wait()

        def row(i, _):
            for c in range(D_MODEL // 16):
                sl = pl.ds(c * 16, 16)
                b1_v[i, sl] = b1_v[i, sl] + b2_v[i, sl]
            return 0

        jax.lax.fori_loop(0, TPW, row, 0)
        pltpu.sync_copy(b1_v, out_hbm.at[pl.ds(base, TPW)])

    return _combine


def kernel(x, w_gate, w1, b1, w2, b2):
    x2d = x.reshape(T, D_MODEL)
    xaug, slots, te_w, tbnt_w = _router(x2d, w_gate)
    slot1 = slots[:, 0].reshape(NW, TPW)
    slot2 = slots[:, 1].reshape(NW, TPW)
    tbnt = tbnt_w[:, 0]
    xg = _sc_dispatch()(xaug, slot1, slot2)
    y = _gmm(tbnt, xg, w1, b1, w2, b2)
    out = _sc_combine()(y, slot1, slot2)
    return out.reshape(x.shape)


# speedup vs baseline: 1.3306x; 1.3306x over previous
import functools
import jax
import jax.numpy as jnp
from jax import lax
from jax.experimental import pallas as pl
from jax.experimental.pallas import tpu as pltpu
from jax.experimental.pallas import tpu_sc as plsc

D_MODEL = 768
N_EXP = 8
D_EXP = 2048
T = 2048
TM = 128
NT = 40
NSLOT = NT * TM
DAUG = D_MODEL + 128
NW = 32
TPW = T // NW
CHUNK = 256


def _router_body(x_ref, wg_ref, xaug_ref, slots_ref, te_ref, tbnt_ref,
                 m_scr, rank_scr):
    x = x_ref[...]
    logits = jnp.dot(x, wg_ref[...], preferred_element_type=jnp.float32)
    cols = jax.lax.broadcasted_iota(jnp.int32, logits.shape, 1)
    big = jnp.int32(N_EXP)
    v1 = jnp.max(logits, axis=1, keepdims=True)
    i1 = jnp.min(jnp.where(logits == v1, cols, big), axis=1, keepdims=True)
    l2 = jnp.where(cols == i1, -jnp.inf, logits)
    v2 = jnp.max(l2, axis=1, keepdims=True)
    i2 = jnp.min(jnp.where(l2 == v2, cols, big), axis=1, keepdims=True)
    e2 = jnp.exp(v2 - v1)
    denom = 1.0 + e2
    g1 = 1.0 / denom
    g2 = e2 / denom

    m = ((cols == i1) | (cols == i2)).astype(jnp.float32)
    m_scr[...] = m

    ri = jax.lax.broadcasted_iota(jnp.int32, (CHUNK, CHUNK), 0)
    ci = jax.lax.broadcasted_iota(jnp.int32, (CHUNK, CHUNK), 1)
    ltri = (ri > ci).astype(jnp.float32)

    def chunk_body(c, carry):
        mc = m_scr[pl.ds(c * CHUNK, CHUNK), :]
        rank_scr[pl.ds(c * CHUNK, CHUNK), :] = (
            jnp.dot(ltri, mc, preferred_element_type=jnp.float32) + carry)
        return carry + jnp.sum(mc, axis=0, keepdims=True)

    counts = jax.lax.fori_loop(0, T // CHUNK, chunk_body,
                               jnp.zeros((1, N_EXP), jnp.float32))

    ntiles = jnp.floor((counts + (TM - 1)) / TM)
    ei = jax.lax.broadcasted_iota(jnp.int32, (N_EXP, N_EXP), 0)
    ej = jax.lax.broadcasted_iota(jnp.int32, (N_EXP, N_EXP), 1)
    strict = (ei < ej).astype(jnp.float32)
    tile_base = jnp.dot(ntiles, strict, preferred_element_type=jnp.float32)
    offset = tile_base * TM

    slot_full = offset + rank_scr[...]
    slot1 = jnp.sum(jnp.where(cols == i1, slot_full, 0.0), axis=1, keepdims=True)
    slot2 = jnp.sum(jnp.where(cols == i2, slot_full, 0.0), axis=1, keepdims=True)

    sc = jax.lax.broadcasted_iota(jnp.int32, (T, 128), 1)
    s1b = jnp.broadcast_to(slot1, (T, 128))
    s2b = jnp.broadcast_to(slot2, (T, 128))
    slots_ref[...] = jnp.where(sc == 0, s1b, jnp.where(sc == 1, s2b, 0.0)
                               ).astype(jnp.int32)

    tj = jax.lax.broadcasted_iota(jnp.int32, (NT, N_EXP), 0).astype(jnp.float32)
    eid = jax.lax.broadcasted_iota(jnp.int32, (NT, N_EXP), 1).astype(jnp.float32)
    tb = jnp.broadcast_to(tile_base, (NT, N_EXP))
    ntb = jnp.broadcast_to(ntiles, (NT, N_EXP))
    ind = ((tj >= tb) & (tj < tb + ntb)).astype(jnp.float32)
    te = jnp.sum(ind * eid, axis=1, keepdims=True)
    te_ref[...] = jnp.broadcast_to(te, (NT, 128)).astype(jnp.int32)

    ri16 = jax.lax.broadcasted_iota(jnp.int32, (16, N_EXP), 0)
    ci16 = jax.lax.broadcasted_iota(jnp.int32, (16, N_EXP), 1)
    tbb = jnp.broadcast_to(tile_base, (16, N_EXP))
    ntbb = jnp.broadcast_to(ntiles, (16, N_EXP))
    isbase = (ri16 < 8).astype(jnp.float32)
    sel = tbb * isbase + ntbb * (1.0 - isbase)
    diag = (ci16 == (ri16 & 7)).astype(jnp.float32)
    tbnt = jnp.sum(diag * sel, axis=1, keepdims=True)
    tbnt_ref[...] = jnp.broadcast_to(tbnt, (16, 128)).astype(jnp.int32)

    xaug_ref[0] = jnp.concatenate(
        [x, jnp.broadcast_to(g1, (T, 128))], axis=1)
    xaug_ref[1] = jnp.concatenate(
        [x, jnp.broadcast_to(g2, (T, 128))], axis=1)


@jax.jit
def _router(x2d, w_gate):
    return pl.pallas_call(
        _router_body,
        in_specs=[pl.BlockSpec((T, D_MODEL), lambda: (0, 0)),
                  pl.BlockSpec((D_MODEL, N_EXP), lambda: (0, 0))],
        out_specs=[pl.BlockSpec((2, T, DAUG), lambda: (0, 0, 0)),
                   pl.BlockSpec((T, 128), lambda: (0, 0)),
                   pl.BlockSpec((NT, 128), lambda: (0, 0)),
                   pl.BlockSpec((16, 128), lambda: (0, 0))],
        out_shape=[jax.ShapeDtypeStruct((2, T, DAUG), jnp.float32),
                   jax.ShapeDtypeStruct((T, 128), jnp.int32),
                   jax.ShapeDtypeStruct((NT, 128), jnp.int32),
                   jax.ShapeDtypeStruct((16, 128), jnp.int32)],
        scratch_shapes=[pltpu.VMEM((T, N_EXP), jnp.float32),
                        pltpu.VMEM((T, N_EXP), jnp.float32)],
    )(x2d, w_gate)


@functools.cache
def _sc_dispatch():
    mesh = plsc.VectorSubcoreMesh(core_axis_name="c", subcore_axis_name="s")

    @functools.partial(
        pl.kernel, mesh=mesh,
        out_type=jax.ShapeDtypeStruct((NSLOT, DAUG), jnp.float32),
        scratch_types=[pltpu.VMEM((TPW,), jnp.int32),
                       pltpu.VMEM((TPW,), jnp.int32),
                       pltpu.VMEM((TPW, DAUG), jnp.float32),
                       pltpu.VMEM((TPW, DAUG), jnp.float32),
                       pltpu.SemaphoreType.DMA,
                       pltpu.SemaphoreType.DMA],
    )
    def _dispatch(xaug_hbm, slot1_hbm, slot2_hbm, xg_hbm, idx1_v, idx2_v,
                  buf1_v, buf2_v, sem1, sem2):
        wid = lax.axis_index("s") * 2 + lax.axis_index("c")
        base = wid * TPW
        pltpu.sync_copy(slot1_hbm.at[wid], idx1_v)
        pltpu.sync_copy(xaug_hbm.at[0, pl.ds(base, TPW)], buf1_v)
        cp1 = pltpu.async_copy(buf1_v, xg_hbm.at[idx1_v], sem1)
        pltpu.sync_copy(slot2_hbm.at[wid], idx2_v)
        pltpu.sync_copy(xaug_hbm.at[1, pl.ds(base, TPW)], buf2_v)
        cp2 = pltpu.async_copy(buf2_v, xg_hbm.at[idx2_v], sem2)
        cp1.wait()
        cp2.wait()

    return _dispatch


def _w_copies(w1_hbm, w2_hbm, w1_buf, w2_buf, wsem, e, slot):
    half = D_EXP // 2
    return [
        pltpu.make_async_copy(w1_hbm.at[e, :, pl.ds(0, half)],
                              w1_buf.at[slot, :, pl.ds(0, half)], wsem),
        pltpu.make_async_copy(w1_hbm.at[e, :, pl.ds(half, half)],
                              w1_buf.at[slot, :, pl.ds(half, half)], wsem),
        pltpu.make_async_copy(w2_hbm.at[e, pl.ds(0, half)],
                              w2_buf.at[slot, pl.ds(0, half)], wsem),
        pltpu.make_async_copy(w2_hbm.at[e, pl.ds(half, half)],
                              w2_buf.at[slot, pl.ds(half, half)], wsem),
    ]


def _gmm_body(tbnt_ref, xg_hbm, w1_hbm, b1_hbm, w2_hbm, b2_hbm, y_hbm,
              w1_buf, w2_buf, b1_buf, b2_buf, xg_buf, y_buf,
              wsems, bsem, xsem, ysem):
    pltpu.make_async_copy(b1_hbm, b1_buf, bsem).start()
    pltpu.make_async_copy(b2_hbm, b2_buf, bsem).start()

    def fetch(e, slot):
        @pl.when(tbnt_ref[8 + e] > 0)
        def _():
            for cp in _w_copies(w1_hbm, w2_hbm, w1_buf, w2_buf,
                                wsems.at[slot], e, slot):
                cp.start()

    fetch(0, 0)
    pltpu.make_async_copy(b1_hbm, b1_buf, bsem).wait()
    pltpu.make_async_copy(b2_hbm, b2_buf, bsem).wait()

    def xg_copy(jg, kp):
        return pltpu.make_async_copy(xg_hbm.at[pl.ds(jg * TM, TM)],
                                     xg_buf.at[kp], xsem)

    def y_copy(jg, kp):
        return pltpu.make_async_copy(y_buf.at[kp],
                                     y_hbm.at[pl.ds(jg * TM, TM)], ysem)

    for e in range(N_EXP):
        slot = e % 2
        if e + 1 < N_EXP:
            fetch(e + 1, (e + 1) % 2)
        nt_e = tbnt_ref[8 + e]
        tb_e = tbnt_ref[e]

        @pl.when(nt_e > 0)
        def _(e=e, slot=slot, nt_e=nt_e, tb_e=tb_e):
            xg_copy(tb_e, 0).start()
            for cp in _w_copies(w1_hbm, w2_hbm, w1_buf, w2_buf,
                                wsems.at[slot], e, slot):
                cp.wait()
            w1 = w1_buf[slot]
            w2 = w2_buf[slot]
            b1 = b1_buf[e, :].reshape(1, D_EXP)
            b2 = b2_buf[e, :].reshape(1, D_MODEL)

            def tile_body(k, _):
                jg = tb_e + k
                kp = jax.lax.rem(k, 2)

                @pl.when(k + 1 < nt_e)
                def _():
                    xg_copy(jg + 1, 1 - kp).start()

                xg_copy(jg, kp).wait()

                @pl.when(k >= 2)
                def _():
                    y_copy(jg - 2, kp).wait()

                xg = xg_buf[kp]
                x = xg[:, :D_MODEL]
                g = xg[:, D_MODEL:D_MODEL + 1]
                h = jnp.maximum(
                    jnp.dot(x, w1, preferred_element_type=jnp.float32) + b1,
                    0.0)
                y_buf[kp] = (jnp.dot(h, w2, preferred_element_type=jnp.float32)
                             + b2) * g
                y_copy(jg, kp).start()
                return 0

            jax.lax.fori_loop(0, nt_e, tile_body, 0)

            @pl.when(nt_e >= 2)
            def _(nt_e=nt_e, tb_e=tb_e):
                y_copy(tb_e + nt_e - 2, jax.lax.rem(nt_e - 2, 2)).wait()

            @pl.when(nt_e >= 1)
            def _(nt_e=nt_e, tb_e=tb_e):
                y_copy(tb_e + nt_e - 1, jax.lax.rem(nt_e - 1, 2)).wait()


@jax.jit
def _gmm(tbnt, xg, w1, b1, w2, b2):
    grid_spec = pltpu.PrefetchScalarGridSpec(
        num_scalar_prefetch=1,
        grid=(1,),
        in_specs=[
            pl.BlockSpec(memory_space=pl.ANY),
            pl.BlockSpec(memory_space=pl.ANY),
            pl.BlockSpec(memory_space=pl.ANY),
            pl.BlockSpec(memory_space=pl.ANY),
            pl.BlockSpec(memory_space=pl.ANY),
        ],
        out_specs=pl.BlockSpec(memory_space=pl.ANY),
        scratch_shapes=[
            pltpu.VMEM((2, D_MODEL, D_EXP), jnp.float32),
            pltpu.VMEM((2, D_EXP, D_MODEL), jnp.float32),
            pltpu.VMEM((N_EXP, D_EXP), jnp.float32),
            pltpu.VMEM((N_EXP, D_MODEL), jnp.float32),
            pltpu.VMEM((2, TM, DAUG), jnp.float32),
            pltpu.VMEM((2, TM, D_MODEL), jnp.float32),
            pltpu.SemaphoreType.DMA((2,)),
            pltpu.SemaphoreType.DMA,
            pltpu.SemaphoreType.DMA,
            pltpu.SemaphoreType.DMA,
        ],
    )
    return pl.pallas_call(
        _gmm_body,
        grid_spec=grid_spec,
        out_shape=jax.ShapeDtypeStruct((NSLOT, D_MODEL), jnp.float32),
    )(tbnt, xg, w1, b1, w2, b2)


@functools.cache
def _sc_combine():
    mesh = plsc.VectorSubcoreMesh(core_axis_name="c", subcore_axis_name="s")

    @functools.partial(
        pl.kernel, mesh=mesh,
        out_type=jax.ShapeDtypeStruct((T, D_MODEL), jnp.float32),
        scratch_types=[pltpu.VMEM((TPW,), jnp.int32),
                       pltpu.VMEM((TPW,), jnp.int32),
                       pltpu.VMEM((TPW, D_MODEL), jnp.float32),
                       pltpu.VMEM((TPW, D_MODEL), jnp.float32),
                       pltpu.SemaphoreType.DMA],
    )
    def _combine(y_hbm, slot1_hbm, slot2_hbm, out_hbm, i1_v, i2_v, b1_v, b2_v,
                 sem):
        wid = lax.axis_index("s") * 2 + lax.axis_index("c")
        base = wid * TPW
        pltpu.sync_copy(slot1_hbm.at[wid], i1_v)
        pltpu.sync_copy(slot2_hbm.at[wid], i2_v)
        pltpu.async_copy(y_hbm.at[i1_v], b1_v, sem).wait()
        pltpu.async_copy(y_hbm.at[i2_v], b2_v, sem).wait()

        def row(i, _):
            for c in range(D_MODEL // 16):
                sl = pl.ds(c * 16, 16)
                b1_v[i, sl] = b1_v[i, sl] + b2_v[i, sl]
            return 0

        jax.lax.fori_loop(0, TPW, row, 0)
        pltpu.sync_copy(b1_v, out_hbm.at[pl.ds(base, TPW)])

    return _combine


def kernel(x, w_gate, w1, b1, w2, b2):
    x2d = x.reshape(T, D_MODEL)
    xaug, slots, te_w, tbnt_w = _router(x2d, w_gate)
    slot1 = slots[:, 0].reshape(NW, TPW)
    slot2 = slots[:, 1].reshape(NW, TPW)
    tbnt = tbnt_w[:, 0]
    xg = _sc_dispatch()(xaug, slot1, slot2)
    y = _gmm(tbnt, xg, w1, b1, w2, b2)
    out = _sc_combine()(y, slot1, slot2)
    return out.reshape(x.shape)
